# baseline (speedup 1.0000x reference)
import functools

import jax
import jax.numpy as jnp
from jax import lax
from jax.experimental import pallas as pl
from jax.experimental.pallas import tpu as pltpu
from jax.experimental.pallas import tpu_sc as plsc

_TOPK = 6
_NE = 64
_BLK = 2048
_BIAS = 1 << 29
_NW = 32
_LANES = 16
_CHUNKS = 4


def _keys_body_t(x_ref, wt_ref, k_ref):
    s = jax.lax.dot_general(
        wt_ref[...], x_ref[...], (((0,), (1,)), ((), ())),
        preferred_element_type=jnp.float32)
    m = jnp.max(s, axis=0, keepdims=True)
    e = jnp.exp(s - m)
    p = e / jnp.sum(e, axis=0, keepdims=True)
    sub = jax.lax.broadcasted_iota(jnp.int32, s.shape, 0)
    pb = jax.lax.bitcast_convert_type(p, jnp.int32)
    k_ref[...] = jax.lax.bitcast_convert_type(
        ((pb & -_NE) | (_NE - 1 - sub)) + _BIAS, jnp.float32)


def _tc_keys_chunk(x, wt, c, nc):
    d = x.shape[1]
    step0 = c * (nc // _BLK)
    return pl.pallas_call(
        _keys_body_t,
        grid=(nc // _BLK,),
        in_specs=[
            pl.BlockSpec((_BLK, d), lambda i: (step0 + i, 0)),
            pl.BlockSpec((d, _NE), lambda i: (0, 0)),
        ],
        out_specs=pl.BlockSpec((_NE, _BLK), lambda i: (0, i)),
        out_shape=jax.ShapeDtypeStruct((_NE, nc), jnp.float32),
        compiler_params=pltpu.CompilerParams(
            dimension_semantics=("parallel",),
        ),
    )(x, wt)


def _make_sc_top6(n):
    tok_w = n // _NW
    groups = tok_w // _LANES
    mesh = plsc.VectorSubcoreMesh(core_axis_name="c", subcore_axis_name="s")

    @functools.partial(
        pl.kernel,
        mesh=mesh,
        out_type=jax.ShapeDtypeStruct((_TOPK, n), jnp.float32),
        scratch_types=[
            pltpu.VMEM((_NE, tok_w), jnp.float32),
            pltpu.VMEM((_TOPK, tok_w), jnp.float32),
            pltpu.SemaphoreType.DMA,
        ],
    )
    def sc_top6(keys_hbm, out_hbm, buf, outbuf, sem):
        wid = lax.axis_index("s") * 2 + lax.axis_index("c")
        base = wid * tok_w
        descs = [
            pltpu.async_copy(keys_hbm.at[e, pl.ds(base, tok_w)], buf.at[e], sem)
            for e in range(_NE)
        ]
        for d_ in descs:
            d_.wait()

        def gbody(g, carry):
            gb = g * _LANES
            t = [jnp.zeros((_LANES,), jnp.float32)] * _TOPK
            for e in range(_NE):
                v = buf[e, pl.ds(gb, _LANES)]
                for j in range(_TOPK - 1):
                    hi = jnp.maximum(t[j], v)
                    v = jnp.minimum(t[j], v)
                    t[j] = hi
                t[_TOPK - 1] = jnp.maximum(t[_TOPK - 1], v)
            for j in range(_TOPK):
                outbuf[j, pl.ds(gb, _LANES)] = t[j]
            return carry

        lax.fori_loop(0, groups, gbody, 0)
        for j in range(_TOPK):
            pltpu.sync_copy(outbuf.at[j], out_hbm.at[j, pl.ds(base, tok_w)])

    return sc_top6


def kernel(x, W):
    n, d = x.shape
    wt = W.T
    nc = n // _CHUNKS
    sc = _make_sc_top6(nc)
    parts = [sc(_tc_keys_chunk(x, wt, c, nc)) for c in range(_CHUNKS)]
    topf = jnp.concatenate(parts, axis=1) if _CHUNKS > 1 else parts[0]
    top = jax.lax.bitcast_convert_type(topf.T, jnp.int32) - _BIAS
    weights = jax.lax.bitcast_convert_type(top & -_NE, jnp.float32)
    indices = _NE - 1 - (top & (_NE - 1))
    return weights, indices


# speedup vs baseline: 1.1505x; 1.1505x over previous
import jax
import jax.numpy as jnp
from jax.experimental import pallas as pl
from jax.experimental.pallas import tpu as pltpu

_TOPK = 6
_NE = 64
_BLK = 2048


def _gate_body_t(x_ref, wt_ref, w_ref, i_ref):
    s = jax.lax.dot_general(
        wt_ref[...], x_ref[...], (((0,), (1,)), ((), ())),
        preferred_element_type=jnp.float32)
    m = jnp.max(s, axis=0, keepdims=True)
    e = jnp.exp(s - m)
    p = e / jnp.sum(e, axis=0, keepdims=True)
    sub = jax.lax.broadcasted_iota(jnp.int32, s.shape, 0)
    pb = jax.lax.bitcast_convert_type(p, jnp.int32)
    key = jax.lax.bitcast_convert_type(
        ((pb & -_NE) | (_NE - 1 - sub)) + (1 << 29), jnp.float32)
    picks = []
    for _ in range(_TOPK):
        km = jnp.max(key, axis=0, keepdims=True)
        picks.append(km)
        key = jnp.where(key == km, -1.0, key)
    top = jax.lax.bitcast_convert_type(
        jnp.concatenate(picks, axis=0), jnp.int32) - (1 << 29)
    top = jax.lax.transpose(top, (1, 0))
    w_ref[...] = jax.lax.bitcast_convert_type(top & -_NE, jnp.float32)
    i_ref[...] = _NE - 1 - (top & (_NE - 1))


def kernel(x, W):
    n, d = x.shape
    wt = W.T
    grid = (n // _BLK,)
    w_t, i_t = pl.pallas_call(
        _gate_body_t,
        grid=grid,
        in_specs=[
            pl.BlockSpec((_BLK, d), lambda i: (i, 0)),
            pl.BlockSpec((d, _NE), lambda i: (0, 0)),
        ],
        out_specs=[
            pl.BlockSpec((_BLK, _TOPK), lambda i: (i, 0)),
            pl.BlockSpec((_BLK, _TOPK), lambda i: (i, 0)),
        ],
        out_shape=[
            jax.ShapeDtypeStruct((n, _TOPK), jnp.float32),
            jax.ShapeDtypeStruct((n, _TOPK), jnp.int32),
        ],
        compiler_params=pltpu.CompilerParams(
            dimension_semantics=("parallel",),
        ),
    )(x, wt)
    return w_t, i_t


# speedup vs baseline: 1.5375x; 1.3363x over previous
import jax
import jax.numpy as jnp
from jax.experimental import pallas as pl
from jax.experimental.pallas import tpu as pltpu

_TOPK = 6
_NE = 64
_BLK = 2048


def _gate_body_t(x_ref, wt_ref, w_ref, i_ref):
    s = jax.lax.dot_general(
        wt_ref[...], x_ref[...], (((0,), (1,)), ((), ())),
        preferred_element_type=jnp.float32)
    m = jnp.max(s, axis=0, keepdims=True)
    e = jnp.exp(s - m)
    p = e / jnp.sum(e, axis=0, keepdims=True)
    sub = jax.lax.broadcasted_iota(jnp.int32, s.shape, 0)
    pb = jax.lax.bitcast_convert_type(p, jnp.int32)
    key = jax.lax.bitcast_convert_type(
        ((pb & -_NE) | (_NE - 1 - sub)) + (1 << 29), jnp.float32)
    picks = []
    for _ in range(_TOPK):
        km = jnp.max(key, axis=0, keepdims=True)
        picks.append(km)
        key = jnp.where(key == km, -1.0, key)
    top = jax.lax.bitcast_convert_type(
        jnp.concatenate(picks, axis=0), jnp.int32) - (1 << 29)
    w_ref[...] = jax.lax.bitcast_convert_type(top & -_NE, jnp.float32)
    i_ref[...] = _NE - 1 - (top & (_NE - 1))


def kernel(x, W):
    n, d = x.shape
    wt = W.T
    grid = (n // _BLK,)
    w_t, i_t = pl.pallas_call(
        _gate_body_t,
        grid=grid,
        in_specs=[
            pl.BlockSpec((_BLK, d), lambda i: (i, 0)),
            pl.BlockSpec((d, _NE), lambda i: (0, 0)),
        ],
        out_specs=[
            pl.BlockSpec((_TOPK, _BLK), lambda i: (0, i)),
            pl.BlockSpec((_TOPK, _BLK), lambda i: (0, i)),
        ],
        out_shape=[
            jax.ShapeDtypeStruct((_TOPK, n), jnp.float32),
            jax.ShapeDtypeStruct((_TOPK, n), jnp.int32),
        ],
        compiler_params=pltpu.CompilerParams(
            dimension_semantics=("parallel",),
        ),
    )(x, wt)
    return w_t.T, i_t.T


# speedup vs baseline: 1.5395x; 1.0013x over previous
import jax
import jax.numpy as jnp
from jax.experimental import pallas as pl
from jax.experimental.pallas import tpu as pltpu

_TOPK = 6
_NE = 64
_BLK = 2048
_BIAS = 1 << 29


def _gate_body_t(x_ref, wt_ref, w_ref, i_ref):
    s = jax.lax.dot_general(
        wt_ref[...], x_ref[...], (((0,), (1,)), ((), ())),
        preferred_element_type=jnp.float32)
    m = jnp.max(s, axis=0, keepdims=True)
    e = jnp.exp(s - m)
    p = e / jnp.sum(e, axis=0, keepdims=True)
    sub = jax.lax.broadcasted_iota(jnp.int32, s.shape, 0)
    pb = jax.lax.bitcast_convert_type(p, jnp.int32)
    key = jax.lax.bitcast_convert_type(
        ((pb & -_NE) | (_NE - 1 - sub)) + _BIAS, jnp.float32)
    picks = []
    for _ in range(_TOPK):
        km = jnp.max(key, axis=0, keepdims=True)
        picks.append(km)
        key = jnp.where(key == km, -1.0, key)
    top = jax.lax.bitcast_convert_type(
        jnp.concatenate(picks, axis=0), jnp.int32) - _BIAS
    w_ref[...] = jax.lax.bitcast_convert_type(top & -_NE, jnp.float32)
    i_ref[...] = _NE - 1 - (top & (_NE - 1))


def kernel(x, W):
    n, d = x.shape
    wt = W.T
    grid = (n // _BLK,)
    w_t, i_t = pl.pallas_call(
        _gate_body_t,
        grid=grid,
        in_specs=[
            pl.BlockSpec((_BLK, d), lambda i: (i, 0)),
            pl.BlockSpec((d, _NE), lambda i: (0, 0)),
        ],
        out_specs=[
            pl.BlockSpec((_TOPK, _BLK), lambda i: (0, i)),
            pl.BlockSpec((_TOPK, _BLK), lambda i: (0, i)),
        ],
        out_shape=[
            jax.ShapeDtypeStruct((_TOPK, n), jnp.float32),
            jax.ShapeDtypeStruct((_TOPK, n), jnp.int32),
        ],
        compiler_params=pltpu.CompilerParams(
            dimension_semantics=("parallel",),
        ),
    )(x, wt)
    return w_t.T, i_t.T


# speedup vs baseline: 1.5861x; 1.0303x over previous
import jax
import jax.numpy as jnp
from jax.experimental import pallas as pl
from jax.experimental.pallas import tpu as pltpu

_TOPK = 6
_NE = 64
_BLK = 2048
_BIAS = 1 << 29


def _gate_body_t(x_ref, wt_ref, w_ref, i_ref):
    s = jax.lax.dot_general(
        wt_ref[...], x_ref[...], (((1,), (1,)), ((), ())),
        preferred_element_type=jnp.float32)
    m = jnp.max(s, axis=0, keepdims=True)
    e = jnp.exp(s - m)
    p = e / jnp.sum(e, axis=0, keepdims=True)
    sub = jax.lax.broadcasted_iota(jnp.int32, s.shape, 0)
    pb = jax.lax.bitcast_convert_type(p, jnp.int32)
    key = jax.lax.bitcast_convert_type(
        ((pb & -_NE) | (_NE - 1 - sub)) + _BIAS, jnp.float32)
    picks = []
    for _ in range(_TOPK):
        km = jnp.max(key, axis=0, keepdims=True)
        picks.append(km)
        key = jnp.where(key == km, -1.0, key)
    top = jax.lax.bitcast_convert_type(
        jnp.concatenate(picks, axis=0), jnp.int32) - _BIAS
    w_ref[...] = jax.lax.bitcast_convert_type(top & -_NE, jnp.float32)
    i_ref[...] = _NE - 1 - (top & (_NE - 1))


def kernel(x, W):
    n, d = x.shape
    wt = W
    grid = (n // _BLK,)
    w_t, i_t = pl.pallas_call(
        _gate_body_t,
        grid=grid,
        in_specs=[
            pl.BlockSpec((_BLK, d), lambda i: (i, 0)),
            pl.BlockSpec((_NE, d), lambda i: (0, 0)),
        ],
        out_specs=[
            pl.BlockSpec((_TOPK, _BLK), lambda i: (0, i)),
            pl.BlockSpec((_TOPK, _BLK), lambda i: (0, i)),
        ],
        out_shape=[
            jax.ShapeDtypeStruct((_TOPK, n), jnp.float32),
            jax.ShapeDtypeStruct((_TOPK, n), jnp.int32),
        ],
        compiler_params=pltpu.CompilerParams(
            dimension_semantics=("parallel",),
        ),
    )(x, wt)
    return w_t.T, i_t.T
